# baseline (speedup 1.0000x reference)
import functools

import jax
import jax.numpy as jnp
import numpy as np
from jax import lax
from jax.experimental import pallas as pl
from jax.experimental.pallas import tpu as pltpu
from jax.experimental.pallas import tpu_sc as plsc

_MAP_SIZES = [64, 32, 16, 8, 4, 2]
_NB = 6
_B = 16
_NOBJ = 64
_NCLS = 21
_SEG_H = 256
_TOTAL = sum(_B * _NB * ms * ms for ms in _MAP_SIZES)
_NSPLIT = 1
_CLS_STEPS = 91
_CLS_RB = _TOTAL // (_NSPLIT * _CLS_STEPS)
_SEG_BH = 64
_SEG_STEPS = _B * (_SEG_H // _SEG_BH)
_K = 12
_STEPS = 96
_NPAD = _STEPS * _K

_LAYER_OFF = [0, 393216, 491520, 516096]
_LAYER_BSTRIDE = [24576, 6144, 1536, 384]


def _dense_body(*refs):
    xs = refs[:_NSPLIT]
    seg_ref, lab_ref = refs[_NSPLIT:_NSPLIT + 2]
    acc_ref = refs[-1]
    i = pl.program_id(0)

    @pl.when(i == 0)
    def _():
        acc_ref[0, 0] = 0.0
        acc_ref[0, 1] = 0.0

    total = jnp.float32(0.0)
    for x_ref in xs:
        x = x_ref[...]
        s = jnp.sum(jnp.exp(x), axis=1)
        total = total + jnp.sum(jnp.log(s)) - jnp.sum(x[:, 0])
    acc_ref[0, 0] += total

    @pl.when(i < _SEG_STEPS)
    def _():
        lab = lab_ref[0]
        x0 = seg_ref[0, 0]
        se = jnp.exp(x0)
        xl = jnp.where(lab == 0, x0, 0.0)
        for c in range(1, _NCLS):
            xc = seg_ref[0, c]
            se = se + jnp.exp(xc)
            xl = jnp.where(lab == c, xc, xl)
        acc_ref[0, 1] += jnp.sum(jnp.log(se)) - jnp.sum(xl)


_CORR_K = 64


def _corr_body(g_ref, *refs):
    cbs = refs[:_CORR_K]
    lbs = refs[_CORR_K:2 * _CORR_K]
    (sub_r, lab_r, win_r, pos_r, t0_r, t1_r, t2_r, t3_r) = \
        refs[2 * _CORR_K:-1]
    acc_ref = refs[-1]
    i = pl.program_id(0)

    @pl.when(i == 0)
    def _():
        for c in range(8):
            acc_ref[0, c] = 0.0

    lane = lax.broadcasted_iota(jnp.int32, (1, _NCLS), 1)
    lane4 = lax.broadcasted_iota(jnp.int32, (1, 4), 1)
    a_cls = jnp.float32(0.0)
    a_loc = jnp.float32(0.0)
    a_n = jnp.float32(0.0)
    for j in range(_CORR_K):
        idx = i * _CORR_K + j
        sub = sub_r[idx]
        lab = lab_r[idx]
        w = win_r[idx]
        p = pos_r[idx]
        x = cbs[j][pl.ds(sub, 1), :]
        x0 = jnp.sum(jnp.where(lane == 0, x, 0.0))
        xl = jnp.sum(jnp.where(lane == lab, x, 0.0))
        a_cls = a_cls + w * (x0 - xl)
        l = lbs[j][pl.ds(sub, 1), :]
        t = jnp.where(lane4 == 0, t0_r[idx],
                      jnp.where(lane4 == 1, t1_r[idx],
                                jnp.where(lane4 == 2, t2_r[idx], t3_r[idx])))
        d = jnp.abs(l - t)
        a_loc = a_loc + p * jnp.sum(jnp.where(d < 1.0, 0.5 * d * d, d - 0.5))
        a_n = a_n + p
    acc_ref[0, 2] += a_cls
    acc_ref[0, 3] += a_loc
    acc_ref[0, 4] += a_n


def _take16(x, idx):
    dnums = lax.GatherDimensionNumbers(
        offset_dims=(), collapsed_slice_dims=(0,), start_index_map=(0,))
    return lax.gather(x, idx[:, None], dnums, slice_sizes=(1,),
                      mode=lax.GatherScatterMode.PROMISE_IN_BOUNDS)


def _sc_body(idxt, clsb, gtt, dft,
             o_g, o_sub, o_lab, o_win, o_pos, o_t0, o_t1, o_t2, o_t3,
             liv, piv, biv, cbv, gtv, dfv,
             s_g, s_sub, s_lab, s_win, s_pos, s_t0, s_t1, s_t2, s_t3,
             zi, zf):
    w = lax.axis_index("s") * 2 + lax.axis_index("c")

    @pl.when(w < _B)
    def _():
        b = w
        pltpu.sync_copy(idxt.at[0, b], liv)
        pltpu.sync_copy(idxt.at[1, b], piv)
        pltpu.sync_copy(idxt.at[2, b], biv)
        pltpu.sync_copy(clsb.at[b], cbv)
        for c in range(4):
            pltpu.sync_copy(gtt.at[c, b], gtv.at[c])
            pltpu.sync_copy(dft.at[c, b], dfv.at[c])

        iota = lax.iota(jnp.int32, 16)
        flats = []
        labs = []
        for v in range(4):
            ly = liv[pl.ds(16 * v, 16)]
            ps = piv[pl.ds(16 * v, 16)]
            bx = biv[pl.ds(16 * v, 16)]
            lb = cbv[pl.ds(16 * v, 16)]
            off = jnp.where(
                ly == 0, _LAYER_OFF[0],
                jnp.where(ly == 1, _LAYER_OFF[1],
                          jnp.where(ly == 2, _LAYER_OFF[2], _LAYER_OFF[3])))
            bst = jnp.where(
                ly == 0, _LAYER_BSTRIDE[0],
                jnp.where(ly == 1, _LAYER_BSTRIDE[1],
                          jnp.where(ly == 2, _LAYER_BSTRIDE[2],
                                    _LAYER_BSTRIDE[3])))
            flats.append(off + b * bst + ps * _NB + bx)
            labs.append(lb)

        for v in range(4):
            dup = jnp.zeros((16,), jnp.bool_)
            for k in range(1, 16):
                rolled = _take16(flats[v], (iota + k) & 15)
                dup = dup | ((rolled == flats[v]) & (iota < 16 - k))
            for u in range(v + 1, 4):
                for k in range(16):
                    rolled = _take16(flats[u], (iota + k) & 15)
                    dup = dup | (rolled == flats[v])
            win = jnp.logical_not(dup)
            f = flats[v]
            sl = pl.ds(16 * v, 16)
            s_g[sl] = f >> 3
            s_sub[sl] = f & 7
            s_lab[sl] = labs[v]
            s_win[sl] = win.astype(jnp.float32)
            s_pos[sl] = (win & (labs[v] > 0)).astype(jnp.float32)
            for c, stc in enumerate((s_t0, s_t1, s_t2, s_t3)):
                gtc = gtv[c, sl]
                dfc = dfv[c, sl]
                stc[sl] = (gtc - dfc) / jnp.float32(0.1)

        base = w * _NOBJ
        outs = (o_g, o_sub, o_lab, o_win, o_pos, o_t0, o_t1, o_t2, o_t3)
        scr = (s_g, s_sub, s_lab, s_win, s_pos, s_t0, s_t1, s_t2, s_t3)
        for o, s in zip(outs, scr):
            pltpu.sync_copy(s, o.at[pl.ds(base, _NOBJ)])

        @pl.when(w == 0)
        def _():
            for t in range(8):
                zi[pl.ds(16 * t, 16)] = jnp.zeros((16,), jnp.int32)
                zf[pl.ds(16 * t, 16)] = jnp.zeros((16,), jnp.float32)
            for o in (o_g, o_sub, o_lab):
                pltpu.sync_copy(zi, o.at[pl.ds(_B * _NOBJ, 128)])
            for o in (o_win, o_pos, o_t0, o_t1, o_t2, o_t3):
                pltpu.sync_copy(zf, o.at[pl.ds(_B * _NOBJ, 128)])


def kernel(Loc, Cls, Seg, gt_box_batch, df_box_batch, idx_batch, cls_batch,
           bat_s, mining, seg_label):
    idxt = jnp.transpose(idx_batch[..., 1:].astype(jnp.int32), (2, 0, 1))
    gtt = jnp.transpose(gt_box_batch, (2, 0, 1))
    dft = jnp.transpose(df_box_batch, (2, 0, 1))
    mesh = plsc.VectorSubcoreMesh(core_axis_name="c", subcore_axis_name="s")
    i32v = jax.ShapeDtypeStruct((_NPAD,), jnp.int32)
    f32v = jax.ShapeDtypeStruct((_NPAD,), jnp.float32)
    gidx, sub, lab, win, pos, t0, t1, t2, t3 = pl.kernel(
        _sc_body,
        mesh=mesh,
        compiler_params=pltpu.CompilerParams(needs_layout_passes=False),
        out_type=(i32v, i32v, i32v, f32v, f32v, f32v, f32v, f32v, f32v),
        scratch_types=(
            [pltpu.VMEM((_NOBJ,), jnp.int32)] * 4
            + [pltpu.VMEM((4, _NOBJ), jnp.float32)] * 2
            + [pltpu.VMEM((_NOBJ,), jnp.int32)] * 3
            + [pltpu.VMEM((_NOBJ,), jnp.float32)] * 6
            + [pltpu.VMEM((128,), jnp.int32),
               pltpu.VMEM((128,), jnp.float32)]
        ),
    )(idxt, cls_batch.astype(jnp.int32), gtt, dft)

    def _seg_i(i):
        j = jnp.minimum(i, _SEG_STEPS - 1)
        return j // (_SEG_H // _SEG_BH), j % (_SEG_H // _SEG_BH)

    def _seg_map(i):
        bi, hi = _seg_i(i)
        return (bi, 0, hi, 0)

    def _lab_map(i):
        bi, hi = _seg_i(i)
        return (bi, hi, 0)

    dacc = pl.pallas_call(
        _dense_body,
        grid=(_CLS_STEPS,),
        in_specs=[
            pl.BlockSpec(
                (_CLS_RB, _NCLS),
                functools.partial(lambda i, s: (i + s * _CLS_STEPS, 0), s=s))
            for s in range(_NSPLIT)
        ] + [
            pl.BlockSpec((1, _NCLS, _SEG_BH, _SEG_H), _seg_map),
            pl.BlockSpec((1, _SEG_BH, _SEG_H), _lab_map),
        ],
        out_specs=pl.BlockSpec((1, 2), lambda i: (0, 0),
                               memory_space=pltpu.SMEM),
        out_shape=jax.ShapeDtypeStruct((1, 2), jnp.float32),
    )(*([Cls] * _NSPLIT), Seg, seg_label.astype(jnp.int32))

    acc = pl.pallas_call(
        _corr_body,
        grid_spec=pltpu.PrefetchScalarGridSpec(
            num_scalar_prefetch=1,
            grid=(_NPAD // _CORR_K,),
            in_specs=[
                pl.BlockSpec(
                    (8, _NCLS),
                    functools.partial(
                        lambda i, g_ref, j: (g_ref[i * _CORR_K + j], 0), j=j))
                for j in range(_CORR_K)
            ] + [
                pl.BlockSpec(
                    (8, 4),
                    functools.partial(
                        lambda i, g_ref, j: (g_ref[i * _CORR_K + j], 0), j=j))
                for j in range(_CORR_K)
            ] + [pl.BlockSpec(memory_space=pltpu.SMEM)] * 8,
            out_specs=pl.BlockSpec((1, 8), lambda i, g_ref: (0, 0),
                                   memory_space=pltpu.SMEM),
        ),
        out_shape=jax.ShapeDtypeStruct((1, 8), jnp.float32),
    )(gidx, *([Cls] * _CORR_K), *([Loc] * _CORR_K),
      sub, lab, win, pos, t0, t1, t2, t3)

    cls_loss = (dacc[0, 0] + acc[0, 2]) / jnp.float32(_TOTAL)
    loc_loss = acc[0, 3] / jnp.maximum(acc[0, 4], 1.0)
    seg_loss = dacc[0, 1] / jnp.float32(_B * _SEG_H * _SEG_H)
    return cls_loss + loc_loss + seg_loss
